# Initial kernel scaffold; baseline (speedup 1.0000x reference)
import jax
import jax.numpy as jnp
from jax.experimental import pallas as pl


def kernel(x, edge_index, edge_type, W_rel1, W_root1, b1, W_rel2, W_root2, b2, Wc1, bc1, Wc2, bc2):
    raise NotImplementedError("write your pallas kernel here")


# speedup vs baseline: 5.8717x; 5.8717x over previous
import functools

import jax
import jax.numpy as jnp
from jax import lax
from jax.experimental import pallas as pl
from jax.experimental.pallas import tpu as pltpu
from jax.experimental.pallas import tpu_sc as plsc

NN = 10000
EE = 320000
FF = 128
HH = 64
RR = 3

NC = 2
NS = 16
NW = NC * NS
KB = 128
NB = 79
EPW = NB * KB
EP = NW * EPW

TROWS = RR * NN + 80
DROW = RR * NN
STRIPE = TROWS // NS
TBROWS = 2 * NN + 8
BN = 1000
L = 16

@functools.cache
def _mesh():
    return plsc.VectorSubcoreMesh(core_axis_name="c", subcore_axis_name="s",
                                  num_cores=NC, num_subcores=NS)


def _enc1_body(x_ref, wr_ref, wroot_ref, b_ref, hcat_ref, root_ref):
    x = x_ref[...]
    for r in range(RR):
        hcat_ref[r] = jnp.dot(x, wr_ref[r], preferred_element_type=jnp.float32)
    root_ref[...] = jnp.dot(x, wroot_ref[...],
                            preferred_element_type=jnp.float32) + b_ref[...]


def _inv_body(c_ref, o_ref):
    c = c_ref[...]
    o_ref[...] = 1.0 / jnp.maximum(c[0] + c[1], 1.0)


def _invcnt(cnts):
    return pl.pallas_call(
        _inv_body,
        out_shape=jax.ShapeDtypeStruct((TROWS, 16), jnp.float32),
    )(cnts)


def _z_from_parts(p_ref, inv_ref, root_ref):
    p = p_ref[...]
    agg = (p[0] + p[1]) * inv_ref[...]
    lo = agg[0, 0] + agg[0, 1] + agg[0, 2]
    hi = agg[1, 0] + agg[1, 1] + agg[1, 2]
    return root_ref[...] + jnp.concatenate([lo, hi], axis=-1)


def _combine1_body(p_ref, inv_ref, root_ref, wr_ref, wroot_ref, b_ref,
                   hnext_ref, rnext_ref):
    z = jnp.maximum(_z_from_parts(p_ref, inv_ref, root_ref), 0.0)
    for r in range(RR):
        hnext_ref[r] = jnp.dot(z, wr_ref[r], preferred_element_type=jnp.float32)
    rnext_ref[...] = jnp.dot(z, wroot_ref[...],
                             preferred_element_type=jnp.float32) + b_ref[...]


def _combine2_body(p_ref, inv_ref, root_ref, wc_ref, bc_ref, t_ref):
    z = _z_from_parts(p_ref, inv_ref, root_ref)
    t_ref[0] = jnp.dot(z, wc_ref[0],
                       preferred_element_type=jnp.float32) + bc_ref[...]
    t_ref[1] = jnp.dot(z, wc_ref[1], preferred_element_type=jnp.float32)


def _enc1(x, wrel, wroot, b):
    return pl.pallas_call(
        _enc1_body,
        out_shape=[jax.ShapeDtypeStruct((RR, NN, HH), jnp.float32),
                   jax.ShapeDtypeStruct((NN, HH), jnp.float32)],
    )(x, wrel, wroot, b)


def _combine1(p, inv, root, wrel2, wroot2, b2):
    grid = NN // BN
    return pl.pallas_call(
        _combine1_body,
        grid=(grid,),
        in_specs=[
            pl.BlockSpec((2, 2, RR, BN, HH // 2), lambda i: (0, 0, 0, i, 0)),
            pl.BlockSpec((RR, BN, 1), lambda i: (0, i, 0)),
            pl.BlockSpec((BN, HH), lambda i: (i, 0)),
            pl.BlockSpec((RR, HH, HH), lambda i: (0, 0, 0)),
            pl.BlockSpec((HH, HH), lambda i: (0, 0)),
            pl.BlockSpec((1, HH), lambda i: (0, 0)),
        ],
        out_specs=[
            pl.BlockSpec((RR, BN, HH), lambda i: (0, i, 0)),
            pl.BlockSpec((BN, HH), lambda i: (i, 0)),
        ],
        out_shape=[jax.ShapeDtypeStruct((RR, NN, HH), jnp.float32),
                   jax.ShapeDtypeStruct((NN, HH), jnp.float32)],
    )(p, inv, root, wrel2, wroot2, b2)


def _combine2(p, inv, root, wc1, bc1):
    grid = NN // BN
    return pl.pallas_call(
        _combine2_body,
        grid=(grid,),
        in_specs=[
            pl.BlockSpec((2, 2, RR, BN, HH // 2), lambda i: (0, 0, 0, i, 0)),
            pl.BlockSpec((RR, BN, 1), lambda i: (0, i, 0)),
            pl.BlockSpec((BN, HH), lambda i: (i, 0)),
            pl.BlockSpec((2, HH, HH), lambda i: (0, 0, 0)),
            pl.BlockSpec((1, HH), lambda i: (0, 0)),
        ],
        out_specs=pl.BlockSpec((2, BN, HH), lambda i: (0, i, 0)),
        out_shape=jax.ShapeDtypeStruct((2, NN, HH), jnp.float32),
    )(p, inv, root, wc1, bc1)


HHH = HH // 2


@functools.cache
def _make_agg():
    @functools.partial(
        pl.kernel,
        out_type=jax.ShapeDtypeStruct((NC, 2, TROWS, HHH), jnp.float32),
        mesh=_mesh(),
        compiler_params=pltpu.CompilerParams(use_tc_tiling_on_sc=False),
        scratch_types=[
            pltpu.VMEM((NB, KB), jnp.int32),
            pltpu.VMEM((NB, KB), jnp.int32),
            pltpu.VMEM((KB, HHH), jnp.float32),
            pltpu.VMEM_SHARED((TROWS, HHH), jnp.float32),
        ],
    )
    def agg(tlo, thi, gidx, sidx, pout, gv, sv, rows, acc):
        c = lax.axis_index("c")
        s = lax.axis_index("s")
        wid = c * NS + s
        z16f = jnp.zeros((L,), jnp.float32)
        base = s * STRIPE

        pltpu.sync_copy(gidx.at[wid], gv)
        pltpu.sync_copy(sidx.at[wid], sv)

        for half, tbl in ((0, tlo), (1, thi)):
            def zrow(r, _):
                for q in range(HHH // L):
                    rows[r, pl.ds(q * L, L)] = z16f
                return 0
            lax.fori_loop(0, KB, zrow, 0)
            for q in range(STRIPE // KB):
                pltpu.sync_copy(rows, acc.at[pl.ds(base + q * KB, KB)])
            rem = STRIPE - (STRIPE // KB) * KB
            if rem:
                pltpu.sync_copy(rows.at[pl.ds(0, rem)],
                                acc.at[pl.ds(base + (STRIPE // KB) * KB, rem)])
            plsc.subcore_barrier()

            def batch(j, _):
                pltpu.sync_copy(tbl.at[gv.at[j]], rows)
                pltpu.sync_copy(rows, acc.at[sv.at[j]], add=True)
                return 0
            lax.fori_loop(0, NB, batch, 0)

            plsc.subcore_barrier()
            pltpu.sync_copy(acc.at[pl.ds(base, STRIPE)],
                            pout.at[c].at[half].at[pl.ds(base, STRIPE)])
            if half == 0:
                plsc.subcore_barrier()

    return agg


CW = 16


@functools.cache
def _make_count():
    @functools.partial(
        pl.kernel,
        out_type=jax.ShapeDtypeStruct((NC, TROWS, CW), jnp.float32),
        mesh=_mesh(),
        compiler_params=pltpu.CompilerParams(use_tc_tiling_on_sc=False),
        scratch_types=[
            pltpu.VMEM((NB, KB), jnp.int32),
            pltpu.VMEM((KB, CW), jnp.float32),
            pltpu.VMEM_SHARED((TROWS, CW), jnp.float32),
        ],
    )
    def count(sidx, cout, sv, buf, acc):
        c = lax.axis_index("c")
        s = lax.axis_index("s")
        wid = c * NS + s
        pltpu.sync_copy(sidx.at[wid], sv)

        def fill(val):
            v = jnp.zeros((L,), jnp.float32) + val
            def frow(r, _):
                buf[r, pl.ds(0, L)] = v
                return 0
            lax.fori_loop(0, KB, frow, 0)

        fill(0.0)
        base = s * STRIPE
        for q in range(STRIPE // KB):
            pltpu.sync_copy(buf, acc.at[pl.ds(base + q * KB, KB)])
        rem = STRIPE - (STRIPE // KB) * KB
        if rem:
            pltpu.sync_copy(buf.at[pl.ds(0, rem)],
                            acc.at[pl.ds(base + (STRIPE // KB) * KB, rem)])
        fill(1.0)
        plsc.subcore_barrier()

        def batch(j, _):
            pltpu.sync_copy(buf, acc.at[sv.at[j]], add=True)
            return 0
        lax.fori_loop(0, NB, batch, 0)

        plsc.subcore_barrier()
        pltpu.sync_copy(acc.at[pl.ds(base, STRIPE)],
                        cout.at[c].at[pl.ds(base, STRIPE)])

    return count


@functools.cache
def _make_decode():
    @functools.partial(
        pl.kernel,
        out_type=jax.ShapeDtypeStruct((EP,), jnp.float32),
        mesh=_mesh(),
        compiler_params=pltpu.CompilerParams(use_tc_tiling_on_sc=False,
                                             needs_layout_passes=False),
        scratch_types=[
            pltpu.VMEM((NB, KB), jnp.int32),
            pltpu.VMEM((NB, KB), jnp.int32),
            pltpu.VMEM((KB, HH), jnp.float32),
            pltpu.VMEM((KB, HH), jnp.float32),
            pltpu.VMEM((KB,), jnp.float32),
            pltpu.VMEM((HH,), jnp.float32),
        ],
    )
    def _decode(tbl, aidx, bidx, w2, out, av, bv, arows, brows, sbuf, wv):
        c = lax.axis_index("c")
        s = lax.axis_index("s")
        wid = c * NS + s
        pltpu.sync_copy(aidx.at[wid], av)
        pltpu.sync_copy(bidx.at[wid], bv)
        pltpu.sync_copy(w2, wv)

        def batch(j, _):
            pltpu.sync_copy(tbl.at[av.at[j]], arows)
            pltpu.sync_copy(tbl.at[bv.at[j]], brows)

            lane = lax.broadcasted_iota(jnp.int32, (L,), 0)

            def edge(e, _):
                acc = jnp.zeros((L,), jnp.float32)
                for q in range(HH // L):
                    a = arows[e, pl.ds(q * L, L)]
                    b = brows[e, pl.ds(q * L, L)]
                    w = wv[pl.ds(q * L, L)]
                    acc = acc + jnp.maximum(a + b, 0.0) * w
                dnums = lax.GatherDimensionNumbers(
                    offset_dims=(), collapsed_slice_dims=(0,),
                    start_index_map=(0,))
                for m in (1, 2, 4, 8):
                    acc = acc + lax.gather(
                        acc, (lane ^ m)[:, None], dnums, (1,),
                        mode=lax.GatherScatterMode.PROMISE_IN_BOUNDS)
                plsc.store_scatter(sbuf, [jnp.zeros((L,), jnp.int32) + e],
                                   acc, mask=lane == 0)
                return 0
            lax.fori_loop(0, KB, edge, 0)
            pltpu.sync_copy(sbuf, out.at[pl.ds(wid * EPW + j * KB, KB)])
            return 0
        lax.fori_loop(0, NB, batch, 0)

    return _decode


def kernel(x, edge_index, edge_type, W_rel1, W_root1, b1,
           W_rel2, W_root2, b2, Wc1, bc1, Wc2, bc2):
    src = edge_index[0]
    dst = edge_index[1]
    et = edge_type
    padt = jnp.full((EP - EE,), DROW, jnp.int32)
    gidx = jnp.concatenate([et * NN + src, padt]).reshape(NW, NB, KB)
    sidx = jnp.concatenate([et * NN + dst, padt]).reshape(NW, NB, KB)
    padd = jnp.full((EP - EE,), 2 * NN, jnp.int32)
    aidx = jnp.concatenate([src, padd]).reshape(NW, NB, KB)
    bidx = jnp.concatenate([NN + dst, padd]).reshape(NW, NB, KB)

    hcat1, root1 = _enc1(x, W_rel1, W_root1, b1.reshape(1, HH))
    hcat1p = jnp.concatenate(
        [hcat1.reshape(RR * NN, HH),
         jnp.zeros((TROWS - RR * NN, HH), jnp.float32)])
    cnts = _make_count()(sidx)
    p1 = _make_agg()(hcat1p[:, :HHH], hcat1p[:, HHH:], gidx, sidx)
    p1r = p1[:, :, :RR * NN].reshape(2, 2, RR, NN, HHH)
    invr = _invcnt(cnts)[:RR * NN, 0].reshape(RR, NN, 1)

    hcat2, root2 = _combine1(p1r, invr, root1, W_rel2, W_root2,
                             b2.reshape(1, HH))
    hcat2p = jnp.concatenate(
        [hcat2.reshape(RR * NN, HH),
         jnp.zeros((TROWS - RR * NN, HH), jnp.float32)])
    p2 = _make_agg()(hcat2p[:, :HHH], hcat2p[:, HHH:], gidx, sidx)
    p2r = p2[:, :, :RR * NN].reshape(2, 2, RR, NN, HHH)

    t = _combine2(p2r, invr, root2, Wc1.reshape(2, HH, HH), bc1.reshape(1, HH))
    tp = jnp.concatenate(
        [t.reshape(2 * NN, HH), jnp.zeros((TBROWS - 2 * NN, HH), jnp.float32)])
    scores = _make_decode()(tp, aidx, bidx, Wc2.reshape(HH))
    return scores[:EE] + bc2[0]
